# baseline (speedup 1.0000x reference)
import functools

import jax
import jax.numpy as jnp
from jax import lax
from jax.experimental import pallas as pl
from jax.experimental.pallas import tpu as pltpu
from jax.experimental.pallas import tpu_sc as plsc

N = 10000
E = 320000
D = 128
HD = 64
SW = 80
H = 8
C = 16
NC = 2
NS = 16
CH = 128
HCH = 64
NCH = 168
G = 24
G2 = 2 * G
NG = NCH // G
EPW = CH * NCH
EP = NS * EPW
RPT = 640
NR = NS * RPT
BR = 1000


def _sc_edge_pass(xl_hbm, xr_hbm, srcg_hbm, dstg_hbm, dsts_hbm, att_hbm,
                  msg_out,
                  att_v, srcg_big, dstg_big, dsts_big,
                  xl_rows0, xr_rows0, xl_rows1, xr_rows1,
                  stage_m0, stage_m1, zero_m, msg_acc,
                  semg0, semg1, semg2, semg3, sems0, sems1, sems2, sems3):
    c = lax.axis_index("c")
    s = lax.axis_index("s")
    xl_rows = (xl_rows0, xl_rows1)
    xr_rows = (xr_rows0, xr_rows1)
    stage_m = (stage_m0, stage_m1)
    semg = (semg0, semg1, semg2, semg3)
    sems = (sems0, sems1, sems2, sems3)
    iota = lax.iota(jnp.int32, 16)
    rbase = s * RPT
    zv = jnp.zeros((16,), jnp.float32)

    def zrow(i, carry):
        for k in range(SW // 16):
            zero_m[i, pl.ds(k * 16, 16)] = zv
        return carry

    lax.fori_loop(0, 64, zrow, 0)

    def zacc(i, carry):
        pltpu.sync_copy(zero_m, msg_acc.at[pl.ds(rbase + i * 64, 64)])
        return carry

    lax.fori_loop(0, RPT // 64, zacc, 0)
    pltpu.sync_copy(att_hbm, att_v)
    plsc.subcore_barrier()

    att = [att_v[pl.ds(c * HD + h * 16, 16)] for h in range(H // NC)]

    def issue_gather(sl, j):
        gb, hf = sl // 2, sl % 2
        pltpu.async_copy(xl_hbm.at[srcg_big.at[j]],
                         xl_rows[gb].at[pl.ds(hf * HCH, HCH)], semg[sl])
        pltpu.async_copy(xr_hbm.at[dstg_big.at[j]],
                         xr_rows[gb].at[pl.ds(hf * HCH, HCH)], semg[sl])

    def wait_gather(sl):
        gb, hf = sl // 2, sl % 2
        pltpu.make_async_copy(xl_hbm.at[srcg_big.at[0]],
                              xl_rows[gb].at[pl.ds(hf * HCH, HCH)],
                              semg[sl]).wait()
        pltpu.make_async_copy(xr_hbm.at[dstg_big.at[0]],
                              xr_rows[gb].at[pl.ds(hf * HCH, HCH)],
                              semg[sl]).wait()

    def wait_scatter(sl):
        gb, hf = sl // 2, sl % 2
        pltpu.make_async_copy(stage_m[gb].at[pl.ds(hf * HCH, HCH)],
                              msg_acc.at[dsts_big.at[0]], sems[sl]).wait()

    def issue_scatter(sl, j):
        gb, hf = sl // 2, sl % 2
        pltpu.async_copy(stage_m[gb].at[pl.ds(hf * HCH, HCH)],
                         msg_acc.at[dsts_big.at[j]], sems[sl], add=True)

    def compute(sl):
        gb, hf = sl // 2, sl % 2

        @plsc.parallel_loop(hf * HCH, hf * HCH + HCH, unroll=8)
        def edge(e):
            usl = jnp.zeros((16,), jnp.float32)
            for h in range(H // NC):
                a = xl_rows[gb][e, pl.ds(h * 16, 16)]
                bb = xr_rows[gb][e, pl.ds(h * 16, 16)]
                v = a + bb
                v = jnp.maximum(v, 0.2 * v)
                lg = jnp.sum(v * att[h])
                u = jnp.exp(jnp.broadcast_to(lg, (16,)))
                stage_m[gb][e, pl.ds(h * 16, 16)] = u * a
                usl = jnp.where(iota == h, u, usl)
            stage_m[gb][e, pl.ds(HD, 16)] = usl

    def group(gl, carry):
        pltpu.sync_copy(srcg_hbm.at[(c * NS + s) * NG + gl], srcg_big)
        pltpu.sync_copy(dstg_hbm.at[(c * NS + s) * NG + gl], dstg_big)
        pltpu.sync_copy(dsts_hbm.at[s * NG + gl], dsts_big)
        for sl in range(4):
            issue_gather(sl, sl)

        def trip(t, tcarry):
            for sl in range(4):
                j = 4 * t + sl
                wait_gather(sl)

                @pl.when(t > 0)
                def _():
                    wait_scatter(sl)

                compute(sl)

                @pl.when(j + 4 < G2)
                def _():
                    issue_gather(sl, j + 4)

                issue_scatter(sl, j)
            return tcarry

        lax.fori_loop(0, G2 // 4, trip, 0)
        for sl in range(4):
            wait_scatter(sl)
        return carry

    lax.fori_loop(0, NG, group, 0)
    plsc.subcore_barrier()
    pltpu.sync_copy(msg_acc.at[pl.ds(rbase, RPT)],
                    msg_out.at[c, pl.ds(rbase, RPT)])


@functools.cache
def _sc_call():
    return pl.kernel(
        _sc_edge_pass,
        out_type=jax.ShapeDtypeStruct((NC, NR, SW), jnp.float32),
        mesh=plsc.VectorSubcoreMesh(core_axis_name="c", subcore_axis_name="s"),
        compiler_params=pltpu.CompilerParams(needs_layout_passes=False,
                                             use_tc_tiling_on_sc=False),
        scratch_types=(
            [pltpu.VMEM((D,), jnp.float32),
             pltpu.VMEM((G2, HCH), jnp.int32),
             pltpu.VMEM((G2, HCH), jnp.int32),
             pltpu.VMEM((G2, HCH), jnp.int32)]
            + 2 * [pltpu.VMEM((CH, HD), jnp.float32),
                   pltpu.VMEM((CH, HD), jnp.float32)]
            + [pltpu.VMEM((CH, SW), jnp.float32),
               pltpu.VMEM((CH, SW), jnp.float32),
               pltpu.VMEM((64, SW), jnp.float32),
               pltpu.VMEM_SHARED((NR, SW), jnp.float32)]
            + 8 * [pltpu.SemaphoreType.DMA]
        ),
    )


def _mm_body(x_ref, w_ref, xl_ref, xr_ref):
    acc = jnp.dot(x_ref[...], w_ref[...], preferred_element_type=jnp.float32)
    xl_ref[0] = acc[:, 0 * HD:1 * HD]
    xl_ref[1] = acc[:, 1 * HD:2 * HD]
    xr_ref[0] = acc[:, 2 * HD:3 * HD]
    xr_ref[1] = acc[:, 3 * HD:4 * HD]


@functools.cache
def _mm_call():
    return pl.pallas_call(
        _mm_body,
        grid=(N // BR,),
        in_specs=[pl.BlockSpec((BR, D), lambda i: (i, 0)),
                  pl.BlockSpec((D, 2 * D), lambda i: (0, 0))],
        out_specs=[pl.BlockSpec((NC, BR, HD), lambda i: (0, i, 0)),
                   pl.BlockSpec((NC, BR, HD), lambda i: (0, i, 0))],
        out_shape=[jax.ShapeDtypeStruct((NC, N, HD), jnp.float32),
                   jax.ShapeDtypeStruct((NC, N, HD), jnp.float32)],
    )


def _norm_block(m_ref, b_ref, bmat_ref):
    m = jnp.concatenate([m_ref[0][:, :HD], m_ref[1][:, :HD]], axis=1)
    u8 = jnp.concatenate([m_ref[0][:, HD:HD + H // NC],
                          m_ref[1][:, HD:HD + H // NC]], axis=1)
    recip = 1.0 / (u8 + 1e-16)
    r128 = jnp.dot(recip, bmat_ref[...], preferred_element_type=jnp.float32)
    return m * r128 + b_ref[...]


def _comb1_body(m_ref, b_ref, bmat_ref, w_ref, xl_ref, xr_ref):
    x1 = _norm_block(m_ref, b_ref, bmat_ref)
    x1 = jnp.where(x1 > 0.0, x1, jnp.exp(x1) - 1.0)
    acc = jnp.dot(x1, w_ref[...], preferred_element_type=jnp.float32)
    xl_ref[0] = acc[:, 0 * HD:1 * HD]
    xl_ref[1] = acc[:, 1 * HD:2 * HD]
    xr_ref[0] = acc[:, 2 * HD:3 * HD]
    xr_ref[1] = acc[:, 3 * HD:4 * HD]


@functools.cache
def _comb1_call():
    return pl.pallas_call(
        _comb1_body,
        grid=(N // BR,),
        in_specs=[pl.BlockSpec((NC, BR, SW), lambda i: (0, i, 0)),
                  pl.BlockSpec((1, D), lambda i: (0, 0)),
                  pl.BlockSpec((H, D), lambda i: (0, 0)),
                  pl.BlockSpec((D, 2 * D), lambda i: (0, 0))],
        out_specs=[pl.BlockSpec((NC, BR, HD), lambda i: (0, i, 0)),
                   pl.BlockSpec((NC, BR, HD), lambda i: (0, i, 0))],
        out_shape=[jax.ShapeDtypeStruct((NC, N, HD), jnp.float32),
                   jax.ShapeDtypeStruct((NC, N, HD), jnp.float32)],
    )


def _comb2_body(m_ref, b_ref, bmat_ref, h_ref, ls_ref):
    h = _norm_block(m_ref, b_ref, bmat_ref)
    mx = jnp.max(h, axis=1, keepdims=True)
    lse = jnp.log(jnp.sum(jnp.exp(h - mx), axis=1, keepdims=True)) + mx
    h_ref[...] = h
    ls_ref[...] = h - lse


@functools.cache
def _comb2_call():
    return pl.pallas_call(
        _comb2_body,
        grid=(N // BR,),
        in_specs=[pl.BlockSpec((NC, BR, SW), lambda i: (0, i, 0)),
                  pl.BlockSpec((1, D), lambda i: (0, 0)),
                  pl.BlockSpec((H, D), lambda i: (0, 0))],
        out_specs=[pl.BlockSpec((BR, D), lambda i: (i, 0)),
                   pl.BlockSpec((BR, D), lambda i: (i, 0))],
        out_shape=[jax.ShapeDtypeStruct((N, D), jnp.float32),
                   jax.ShapeDtypeStruct((N, D), jnp.float32)],
    )


def kernel(x, edge_index, W_l1, W_r1, att1, b1, W_l2, W_r2, att2, b2):
    loops = jnp.arange(N, dtype=jnp.int32)
    src = jnp.concatenate([edge_index[0], loops])
    dst = jnp.concatenate([edge_index[1], loops])
    pad = EP - (E + N)
    zpad = jnp.zeros((pad,), jnp.int32)
    core_off = jnp.array([0, N], jnp.int32).reshape(NC, 1, 1, 1)
    srcg = (jnp.concatenate([src, zpad]).reshape(1, NS, NCH, CH)
            + core_off).reshape(NC * NS * NG, G2, HCH)
    dstg = (jnp.concatenate([dst, zpad]).reshape(1, NS, NCH, CH)
            + core_off).reshape(NC * NS * NG, G2, HCH)
    dsts = jnp.concatenate(
        [dst, jnp.full((pad,), N, jnp.int32)]).reshape(NS * NG, G2, HCH)

    bmat = jnp.repeat(jnp.eye(H, dtype=jnp.float32), C, axis=1)

    Wc1 = jnp.concatenate([W_l1, W_r1], axis=1)
    Wc2 = jnp.concatenate([W_l2, W_r2], axis=1)

    xl1, xr1 = _mm_call()(x, Wc1)
    msg1 = _sc_call()(xl1.reshape(NC * N, HD), xr1.reshape(NC * N, HD),
                      srcg, dstg, dsts, att1.reshape(-1))
    xl2, xr2 = _comb1_call()(msg1, b1.reshape(1, D), bmat, Wc2)
    msg2 = _sc_call()(xl2.reshape(NC * N, HD), xr2.reshape(NC * N, HD),
                      srcg, dstg, dsts, att2.reshape(-1))
    h, ls = _comb2_call()(msg2, b2.reshape(1, D), bmat)
    return (h, ls)


# speedup vs baseline: 1.3391x; 1.3391x over previous
import functools

import jax
import jax.numpy as jnp
from jax import lax
from jax.experimental import pallas as pl
from jax.experimental.pallas import tpu as pltpu
from jax.experimental.pallas import tpu_sc as plsc

N = 10000
E = 320000
D = 128
HD = 64
H = 8
C = 16
NC = 2
NS = 16
CH = 128
NCH = 162
EPW = CH * NCH
EP = NS * EPW
RPT = 640
NR = NS * RPT
BR = 1000


def _sc_edge_pass(xl_hbm, xr_hbm, srcg_hbm, dstg_hbm, dsts_hbm, att_hbm,
                  msg_out, u_out,
                  att_v, srcg_v0, dstg_v0, dsts_v0, xl_rows0, xr_rows0,
                  stage_m0, stage_u0, srcg_v1, dstg_v1, dsts_v1, xl_rows1,
                  xr_rows1, stage_m1, stage_u1, zero_m, zero_u,
                  msg_acc, u_acc, semg0, semg1, sems0, sems1):
    c = lax.axis_index("c")
    s = lax.axis_index("s")
    srcg = (srcg_v0, srcg_v1)
    dstg = (dstg_v0, dstg_v1)
    dsts = (dsts_v0, dsts_v1)
    xl_rows = (xl_rows0, xl_rows1)
    xr_rows = (xr_rows0, xr_rows1)
    stage_m = (stage_m0, stage_m1)
    stage_u = (stage_u0, stage_u1)
    semg = (semg0, semg1)
    sems = (sems0, sems1)

    zv = jnp.zeros((16,), jnp.float32)

    def zrow(i, carry):
        for k in range(HD // 16):
            zero_m[i, pl.ds(k * 16, 16)] = zv
        zero_u[i, :] = zv
        return carry

    lax.fori_loop(0, 64, zrow, 0)
    rbase = s * RPT

    def zacc(i, carry):
        pltpu.sync_copy(zero_m, msg_acc.at[pl.ds(rbase + i * 64, 64)])
        pltpu.sync_copy(zero_u, u_acc.at[pl.ds(rbase + i * 64, 64)])
        return carry

    lax.fori_loop(0, RPT // 64, zacc, 0)
    pltpu.sync_copy(att_hbm, att_v)
    plsc.subcore_barrier()

    att = [att_v[pl.ds(c * HD + h * 16, 16)] for h in range(H // NC)]
    iota = lax.iota(jnp.int32, 16)
    row_off = jnp.broadcast_to(c * N, (16,)).astype(jnp.int32)
    ebase0 = s * EPW

    def issue_gather(p, eb):
        pltpu.sync_copy(srcg_hbm.at[pl.ds(eb, CH)], srcg[p])
        pltpu.sync_copy(dstg_hbm.at[pl.ds(eb, CH)], dstg[p])
        for j in range(CH // 16):
            sl = pl.ds(j * 16, 16)
            srcg[p][sl] = srcg[p][sl] + row_off
            dstg[p][sl] = dstg[p][sl] + row_off
        pltpu.async_copy(xl_hbm.at[srcg[p]], xl_rows[p], semg[p])
        pltpu.async_copy(xr_hbm.at[dstg[p]], xr_rows[p], semg[p])

    def wait_gather(p):
        pltpu.make_async_copy(xl_hbm.at[srcg[p]], xl_rows[p], semg[p]).wait()
        pltpu.make_async_copy(xr_hbm.at[dstg[p]], xr_rows[p], semg[p]).wait()

    def wait_scatter(p):
        pltpu.make_async_copy(stage_m[p], msg_acc.at[dsts[p]], sems[p]).wait()
        pltpu.make_async_copy(stage_u[p], u_acc.at[dsts[p]], sems[p]).wait()

    def compute(p):
        @plsc.parallel_loop(0, CH, unroll=8)
        def edge(e):
            usl = jnp.zeros((16,), jnp.float32)
            for h in range(H // NC):
                a = xl_rows[p][e, pl.ds(h * 16, 16)]
                bb = xr_rows[p][e, pl.ds(h * 16, 16)]
                v = a + bb
                v = jnp.maximum(v, 0.2 * v)
                lg = jnp.sum(v * att[h])
                u = jnp.exp(jnp.broadcast_to(lg, (16,)))
                stage_m[p][e, pl.ds(h * 16, 16)] = u * a
                usl = jnp.where(iota == h, u, usl)
            stage_u[p][e, :] = usl

    def issue_scatter(p, eb):
        pltpu.sync_copy(dsts_hbm.at[pl.ds(eb, CH)], dsts[p])
        pltpu.async_copy(stage_m[p], msg_acc.at[dsts[p]], sems[p], add=True)
        pltpu.async_copy(stage_u[p], u_acc.at[dsts[p]], sems[p], add=True)

    nsup = NCH // 2
    issue_gather(0, ebase0)

    def sup(k2, carry):
        c0 = ebase0 + (2 * k2) * CH
        issue_gather(1, c0 + CH)
        wait_gather(0)

        @pl.when(k2 > 0)
        def _():
            wait_scatter(0)

        compute(0)
        issue_scatter(0, c0)

        @pl.when(k2 < nsup - 1)
        def _():
            issue_gather(0, c0 + 2 * CH)

        wait_gather(1)

        @pl.when(k2 > 0)
        def _():
            wait_scatter(1)

        compute(1)
        issue_scatter(1, c0 + CH)
        return carry

    lax.fori_loop(0, nsup, sup, 0)
    wait_scatter(0)
    wait_scatter(1)
    plsc.subcore_barrier()
    pltpu.sync_copy(msg_acc.at[pl.ds(rbase, RPT)],
                    msg_out.at[c, pl.ds(rbase, RPT)])
    pltpu.sync_copy(u_acc.at[pl.ds(rbase, RPT)],
                    u_out.at[c, pl.ds(rbase, RPT)])


@functools.cache
def _sc_call():
    return pl.kernel(
        _sc_edge_pass,
        out_type=[jax.ShapeDtypeStruct((NC, NR, HD), jnp.float32),
                  jax.ShapeDtypeStruct((NC, NR, 16), jnp.float32)],
        mesh=plsc.VectorSubcoreMesh(core_axis_name="c", subcore_axis_name="s"),
        compiler_params=pltpu.CompilerParams(needs_layout_passes=False,
                                             use_tc_tiling_on_sc=False),
        scratch_types=(
            [pltpu.VMEM((D,), jnp.float32)]
            + 2 * [pltpu.VMEM((CH,), jnp.int32),
                   pltpu.VMEM((CH,), jnp.int32),
                   pltpu.VMEM((CH,), jnp.int32),
                   pltpu.VMEM((CH, HD), jnp.float32),
                   pltpu.VMEM((CH, HD), jnp.float32),
                   pltpu.VMEM((CH, HD), jnp.float32),
                   pltpu.VMEM((CH, 16), jnp.float32)]
            + [pltpu.VMEM((64, HD), jnp.float32),
               pltpu.VMEM((64, 16), jnp.float32),
               pltpu.VMEM_SHARED((NR, HD), jnp.float32),
               pltpu.VMEM_SHARED((NR, 16), jnp.float32)]
            + 4 * [pltpu.SemaphoreType.DMA]
        ),
    )


def _mm_body(x_ref, w_ref, xl_ref, xr_ref):
    acc = jnp.dot(x_ref[...], w_ref[...], preferred_element_type=jnp.float32)
    xl_ref[0] = acc[:, 0 * HD:1 * HD]
    xl_ref[1] = acc[:, 1 * HD:2 * HD]
    xr_ref[0] = acc[:, 2 * HD:3 * HD]
    xr_ref[1] = acc[:, 3 * HD:4 * HD]


@functools.cache
def _mm_call():
    return pl.pallas_call(
        _mm_body,
        grid=(N // BR,),
        in_specs=[pl.BlockSpec((BR, D), lambda i: (i, 0)),
                  pl.BlockSpec((D, 2 * D), lambda i: (0, 0))],
        out_specs=[pl.BlockSpec((NC, BR, HD), lambda i: (0, i, 0)),
                   pl.BlockSpec((NC, BR, HD), lambda i: (0, i, 0))],
        out_shape=[jax.ShapeDtypeStruct((NC, N, HD), jnp.float32),
                   jax.ShapeDtypeStruct((NC, N, HD), jnp.float32)],
    )


def _norm_block(m_ref, u_ref, b_ref, bmat_ref):
    m = jnp.concatenate([m_ref[0], m_ref[1]], axis=1)
    u8 = jnp.concatenate([u_ref[0][:, :H // NC], u_ref[1][:, :H // NC]],
                         axis=1)
    recip = 1.0 / (u8 + 1e-16)
    r128 = jnp.dot(recip, bmat_ref[...], preferred_element_type=jnp.float32)
    return m * r128 + b_ref[...]


def _comb1_body(m_ref, u_ref, b_ref, bmat_ref, w_ref, xl_ref, xr_ref):
    x1 = _norm_block(m_ref, u_ref, b_ref, bmat_ref)
    x1 = jnp.where(x1 > 0.0, x1, jnp.exp(x1) - 1.0)
    acc = jnp.dot(x1, w_ref[...], preferred_element_type=jnp.float32)
    xl_ref[0] = acc[:, 0 * HD:1 * HD]
    xl_ref[1] = acc[:, 1 * HD:2 * HD]
    xr_ref[0] = acc[:, 2 * HD:3 * HD]
    xr_ref[1] = acc[:, 3 * HD:4 * HD]


@functools.cache
def _comb1_call():
    return pl.pallas_call(
        _comb1_body,
        grid=(N // BR,),
        in_specs=[pl.BlockSpec((NC, BR, HD), lambda i: (0, i, 0)),
                  pl.BlockSpec((NC, BR, 16), lambda i: (0, i, 0)),
                  pl.BlockSpec((1, D), lambda i: (0, 0)),
                  pl.BlockSpec((H, D), lambda i: (0, 0)),
                  pl.BlockSpec((D, 2 * D), lambda i: (0, 0))],
        out_specs=[pl.BlockSpec((NC, BR, HD), lambda i: (0, i, 0)),
                   pl.BlockSpec((NC, BR, HD), lambda i: (0, i, 0))],
        out_shape=[jax.ShapeDtypeStruct((NC, N, HD), jnp.float32),
                   jax.ShapeDtypeStruct((NC, N, HD), jnp.float32)],
    )


def _comb2_body(m_ref, u_ref, b_ref, bmat_ref, h_ref, ls_ref):
    h = _norm_block(m_ref, u_ref, b_ref, bmat_ref)
    mx = jnp.max(h, axis=1, keepdims=True)
    lse = jnp.log(jnp.sum(jnp.exp(h - mx), axis=1, keepdims=True)) + mx
    h_ref[...] = h
    ls_ref[...] = h - lse


@functools.cache
def _comb2_call():
    return pl.pallas_call(
        _comb2_body,
        grid=(N // BR,),
        in_specs=[pl.BlockSpec((NC, BR, HD), lambda i: (0, i, 0)),
                  pl.BlockSpec((NC, BR, 16), lambda i: (0, i, 0)),
                  pl.BlockSpec((1, D), lambda i: (0, 0)),
                  pl.BlockSpec((H, D), lambda i: (0, 0))],
        out_specs=[pl.BlockSpec((BR, D), lambda i: (i, 0)),
                   pl.BlockSpec((BR, D), lambda i: (i, 0))],
        out_shape=[jax.ShapeDtypeStruct((N, D), jnp.float32),
                   jax.ShapeDtypeStruct((N, D), jnp.float32)],
    )


def kernel(x, edge_index, W_l1, W_r1, att1, b1, W_l2, W_r2, att2, b2):
    loops = jnp.arange(N, dtype=jnp.int32)
    src = jnp.concatenate([edge_index[0], loops])
    dst = jnp.concatenate([edge_index[1], loops])
    pad = EP - (E + N)
    zpad = jnp.zeros((pad,), jnp.int32)
    srcg = jnp.concatenate([src, zpad])
    dstg = jnp.concatenate([dst, zpad])
    dsts = jnp.concatenate([dst, jnp.full((pad,), N, jnp.int32)])

    bmat = jnp.repeat(jnp.eye(H, dtype=jnp.float32), C, axis=1)

    Wc1 = jnp.concatenate([W_l1, W_r1], axis=1)
    Wc2 = jnp.concatenate([W_l2, W_r2], axis=1)

    xl1, xr1 = _mm_call()(x, Wc1)
    msg1, u1 = _sc_call()(xl1.reshape(NC * N, HD), xr1.reshape(NC * N, HD),
                          srcg, dstg, dsts, att1.reshape(-1))
    xl2, xr2 = _comb1_call()(msg1, u1, b1.reshape(1, D), bmat, Wc2)
    msg2, u2 = _sc_call()(xl2.reshape(NC * N, HD), xr2.reshape(NC * N, HD),
                          srcg, dstg, dsts, att2.reshape(-1))
    h, ls = _comb2_call()(msg2, u2, b2.reshape(1, D), bmat)
    return (h, ls)


# speedup vs baseline: 1.5534x; 1.1600x over previous
import functools

import jax
import jax.numpy as jnp
from jax import lax
from jax.experimental import pallas as pl
from jax.experimental.pallas import tpu as pltpu
from jax.experimental.pallas import tpu_sc as plsc

N = 10000
E = 320000
D = 128
HD = 64
H = 8
C = 16
NC = 2
NS = 16
CH = 128
NCH = 162
EPW = CH * NCH
EP = NS * EPW
OFFN = N + 8
RPT = 640
NR = NS * RPT
BR = 1000


def _sc_edge_pass(xl_hbm, xr_hbm, srcg_hbm, dstg_hbm, att_hbm,
                  msg_out, u_out,
                  att_v, srcg_v0, dstg_v0, dsts_v0, xl_rows0, xr_rows0,
                  stage_m0, stage_u0, srcg_v1, dstg_v1, dsts_v1, xl_rows1,
                  xr_rows1, stage_m1, stage_u1, zero_m, zero_u,
                  msg_acc, u_acc, semg0, semg1, sems0, sems1):
    c = lax.axis_index("c")
    s = lax.axis_index("s")
    srcg = (srcg_v0, srcg_v1)
    dstg = (dstg_v0, dstg_v1)
    dsts = (dsts_v0, dsts_v1)
    xl_rows = (xl_rows0, xl_rows1)
    xr_rows = (xr_rows0, xr_rows1)
    stage_m = (stage_m0, stage_m1)
    stage_u = (stage_u0, stage_u1)
    semg = (semg0, semg1)
    sems = (sems0, sems1)

    zv = jnp.zeros((16,), jnp.float32)

    def zrow(i, carry):
        for k in range(HD // 16):
            zero_m[i, pl.ds(k * 16, 16)] = zv
        zero_u[i, :] = zv
        return carry

    lax.fori_loop(0, 64, zrow, 0)
    rbase = s * RPT

    def zacc(i, carry):
        pltpu.sync_copy(zero_m, msg_acc.at[pl.ds(rbase + i * 64, 64)])
        pltpu.sync_copy(zero_u, u_acc.at[pl.ds(rbase + i * 64, 64)])
        return carry

    lax.fori_loop(0, RPT // 64, zacc, 0)
    pltpu.sync_copy(att_hbm, att_v)
    plsc.subcore_barrier()

    att = [att_v[pl.ds(c * HD + h * 16, 16)] for h in range(H // NC)]
    iota = lax.iota(jnp.int32, 16)
    row_off = jnp.broadcast_to(c * OFFN, (16,)).astype(jnp.int32)
    ebase0 = s * EPW

    def issue_gather(p, eb):
        pltpu.sync_copy(srcg_hbm.at[pl.ds(eb, CH)], srcg[p])
        pltpu.sync_copy(dstg_hbm.at[pl.ds(eb, CH)], dstg[p])
        for j in range(CH // 16):
            sl = pl.ds(j * 16, 16)
            srcg[p][sl] = srcg[p][sl] + row_off
            dstg[p][sl] = dstg[p][sl] + row_off
        pltpu.async_copy(xl_hbm.at[srcg[p]], xl_rows[p], semg[p])
        pltpu.async_copy(xr_hbm.at[dstg[p]], xr_rows[p], semg[p])

    def wait_gather(p):
        pltpu.make_async_copy(xl_hbm.at[srcg[p]], xl_rows[p], semg[p]).wait()
        pltpu.make_async_copy(xr_hbm.at[dstg[p]], xr_rows[p], semg[p]).wait()

    def wait_scatter(p):
        pltpu.make_async_copy(stage_m[p], msg_acc.at[dsts[p]], sems[p]).wait()
        pltpu.make_async_copy(stage_u[p], u_acc.at[dsts[p]], sems[p]).wait()

    def compute(p):
        @plsc.parallel_loop(0, CH, unroll=8)
        def edge(e):
            usl = jnp.zeros((16,), jnp.float32)
            for h in range(H // NC):
                a = xl_rows[p][e, pl.ds(h * 16, 16)]
                bb = xr_rows[p][e, pl.ds(h * 16, 16)]
                v = a + bb
                v = jnp.maximum(v, 0.2 * v)
                lg = jnp.sum(v * att[h])
                u = jnp.exp(jnp.broadcast_to(lg, (16,)))
                stage_m[p][e, pl.ds(h * 16, 16)] = u * a
                usl = jnp.where(iota == h, u, usl)
            stage_u[p][e, :] = usl

    def issue_scatter(p, eb):
        for j in range(CH // 16):
            sl = pl.ds(j * 16, 16)
            dsts[p][sl] = dstg[p][sl] - row_off
        pltpu.async_copy(stage_m[p], msg_acc.at[dsts[p]], sems[p], add=True)
        pltpu.async_copy(stage_u[p], u_acc.at[dsts[p]], sems[p], add=True)

    nsup = NCH // 2
    issue_gather(0, ebase0)

    def sup(k2, carry):
        c0 = ebase0 + (2 * k2) * CH
        issue_gather(1, c0 + CH)
        wait_gather(0)

        @pl.when(k2 > 0)
        def _():
            wait_scatter(0)

        compute(0)
        issue_scatter(0, c0)

        @pl.when(k2 < nsup - 1)
        def _():
            issue_gather(0, c0 + 2 * CH)

        wait_gather(1)

        @pl.when(k2 > 0)
        def _():
            wait_scatter(1)

        compute(1)
        issue_scatter(1, c0 + CH)
        return carry

    lax.fori_loop(0, nsup, sup, 0)
    wait_scatter(0)
    wait_scatter(1)
    plsc.subcore_barrier()
    pltpu.sync_copy(msg_acc.at[pl.ds(rbase, RPT)],
                    msg_out.at[c, pl.ds(rbase, RPT)])
    pltpu.sync_copy(u_acc.at[pl.ds(rbase, RPT)],
                    u_out.at[c, pl.ds(rbase, RPT)])


@functools.cache
def _sc_call():
    return pl.kernel(
        _sc_edge_pass,
        out_type=[jax.ShapeDtypeStruct((NC, NR, HD), jnp.float32),
                  jax.ShapeDtypeStruct((NC, NR, 16), jnp.float32)],
        mesh=plsc.VectorSubcoreMesh(core_axis_name="c", subcore_axis_name="s"),
        compiler_params=pltpu.CompilerParams(needs_layout_passes=False,
                                             use_tc_tiling_on_sc=False),
        scratch_types=(
            [pltpu.VMEM((D,), jnp.float32)]
            + 2 * [pltpu.VMEM((CH,), jnp.int32),
                   pltpu.VMEM((CH,), jnp.int32),
                   pltpu.VMEM((CH,), jnp.int32),
                   pltpu.VMEM((CH, HD), jnp.float32),
                   pltpu.VMEM((CH, HD), jnp.float32),
                   pltpu.VMEM((CH, HD), jnp.float32),
                   pltpu.VMEM((CH, 16), jnp.float32)]
            + [pltpu.VMEM((64, HD), jnp.float32),
               pltpu.VMEM((64, 16), jnp.float32),
               pltpu.VMEM_SHARED((NR, HD), jnp.float32),
               pltpu.VMEM_SHARED((NR, 16), jnp.float32)]
            + 4 * [pltpu.SemaphoreType.DMA]
        ),
    )


def _mm_body(x_ref, w_ref, xl_ref, xr_ref):
    acc = jnp.dot(x_ref[...], w_ref[...], preferred_element_type=jnp.float32)
    xl_ref[0] = acc[:, 0 * HD:1 * HD]
    xl_ref[1] = acc[:, 1 * HD:2 * HD]
    xr_ref[0] = acc[:, 2 * HD:3 * HD]
    xr_ref[1] = acc[:, 3 * HD:4 * HD]


@functools.cache
def _mm_call():
    return pl.pallas_call(
        _mm_body,
        grid=(N // BR,),
        in_specs=[pl.BlockSpec((BR, D), lambda i: (i, 0)),
                  pl.BlockSpec((D, 2 * D), lambda i: (0, 0))],
        out_specs=[pl.BlockSpec((NC, BR, HD), lambda i: (0, i, 0)),
                   pl.BlockSpec((NC, BR, HD), lambda i: (0, i, 0))],
        out_shape=[jax.ShapeDtypeStruct((NC, OFFN, HD), jnp.float32),
                   jax.ShapeDtypeStruct((NC, OFFN, HD), jnp.float32)],
    )


def _norm_block(m_ref, u_ref, b_ref, bmat_ref):
    m = jnp.concatenate([m_ref[0], m_ref[1]], axis=1)
    u8 = jnp.concatenate([u_ref[0][:, :H // NC], u_ref[1][:, :H // NC]],
                         axis=1)
    recip = 1.0 / (u8 + 1e-16)
    r128 = jnp.dot(recip, bmat_ref[...], preferred_element_type=jnp.float32)
    return m * r128 + b_ref[...]


def _comb1_body(m_ref, u_ref, b_ref, bmat_ref, w_ref, xl_ref, xr_ref):
    x1 = _norm_block(m_ref, u_ref, b_ref, bmat_ref)
    x1 = jnp.where(x1 > 0.0, x1, jnp.exp(x1) - 1.0)
    acc = jnp.dot(x1, w_ref[...], preferred_element_type=jnp.float32)
    xl_ref[0] = acc[:, 0 * HD:1 * HD]
    xl_ref[1] = acc[:, 1 * HD:2 * HD]
    xr_ref[0] = acc[:, 2 * HD:3 * HD]
    xr_ref[1] = acc[:, 3 * HD:4 * HD]


@functools.cache
def _comb1_call():
    return pl.pallas_call(
        _comb1_body,
        grid=(N // BR,),
        in_specs=[pl.BlockSpec((NC, BR, HD), lambda i: (0, i, 0)),
                  pl.BlockSpec((NC, BR, 16), lambda i: (0, i, 0)),
                  pl.BlockSpec((1, D), lambda i: (0, 0)),
                  pl.BlockSpec((H, D), lambda i: (0, 0)),
                  pl.BlockSpec((D, 2 * D), lambda i: (0, 0))],
        out_specs=[pl.BlockSpec((NC, BR, HD), lambda i: (0, i, 0)),
                   pl.BlockSpec((NC, BR, HD), lambda i: (0, i, 0))],
        out_shape=[jax.ShapeDtypeStruct((NC, OFFN, HD), jnp.float32),
                   jax.ShapeDtypeStruct((NC, OFFN, HD), jnp.float32)],
    )


def _comb2_body(m_ref, u_ref, b_ref, bmat_ref, h_ref, ls_ref):
    h = _norm_block(m_ref, u_ref, b_ref, bmat_ref)
    mx = jnp.max(h, axis=1, keepdims=True)
    lse = jnp.log(jnp.sum(jnp.exp(h - mx), axis=1, keepdims=True)) + mx
    h_ref[...] = h
    ls_ref[...] = h - lse


@functools.cache
def _comb2_call():
    return pl.pallas_call(
        _comb2_body,
        grid=(N // BR,),
        in_specs=[pl.BlockSpec((NC, BR, HD), lambda i: (0, i, 0)),
                  pl.BlockSpec((NC, BR, 16), lambda i: (0, i, 0)),
                  pl.BlockSpec((1, D), lambda i: (0, 0)),
                  pl.BlockSpec((H, D), lambda i: (0, 0))],
        out_specs=[pl.BlockSpec((BR, D), lambda i: (i, 0)),
                   pl.BlockSpec((BR, D), lambda i: (i, 0))],
        out_shape=[jax.ShapeDtypeStruct((N, D), jnp.float32),
                   jax.ShapeDtypeStruct((N, D), jnp.float32)],
    )


def kernel(x, edge_index, W_l1, W_r1, att1, b1, W_l2, W_r2, att2, b2):
    loops = jnp.arange(N, dtype=jnp.int32)
    src = jnp.concatenate([edge_index[0], loops])
    dst = jnp.concatenate([edge_index[1], loops])
    pad = EP - (E + N)
    zpad = jnp.zeros((pad,), jnp.int32)
    srcg = jnp.concatenate([src, zpad])
    dstg = jnp.concatenate([dst, jnp.full((pad,), N, jnp.int32)])

    bmat = jnp.repeat(jnp.eye(H, dtype=jnp.float32), C, axis=1)

    Wc1 = jnp.concatenate([W_l1, W_r1], axis=1)
    Wc2 = jnp.concatenate([W_l2, W_r2], axis=1)

    xl1, xr1 = _mm_call()(x, Wc1)
    msg1, u1 = _sc_call()(xl1.reshape(NC * OFFN, HD),
                          xr1.reshape(NC * OFFN, HD),
                          srcg, dstg, att1.reshape(-1))
    xl2, xr2 = _comb1_call()(msg1, u1, b1.reshape(1, D), bmat, Wc2)
    msg2, u2 = _sc_call()(xl2.reshape(NC * OFFN, HD),
                          xr2.reshape(NC * OFFN, HD),
                          srcg, dstg, att2.reshape(-1))
    h, ls = _comb2_call()(msg2, u2, b2.reshape(1, D), bmat)
    return (h, ls)


# speedup vs baseline: 1.7884x; 1.1513x over previous
import functools

import jax
import jax.numpy as jnp
from jax import lax
from jax.experimental import pallas as pl
from jax.experimental.pallas import tpu as pltpu
from jax.experimental.pallas import tpu_sc as plsc

N = 10000
E = 320000
D = 128
HD = 64
H = 8
C = 16
NC = 2
NS = 16
CH = 128
NCH = 162
EPW = CH * NCH
EP = NS * EPW
OFFN = N + 8
RPT = 640
NR = NS * RPT
BR = 1000


def _sc_edge_pass(xl_hbm, xr_hbm, sd_hbm, att_hbm,
                  msg_out, u_out,
                  att_v, sd_v0, dsts_v0, xl_rows0, xr_rows0,
                  stage_m0, stage_u0, sd_v1, dsts_v1, xl_rows1,
                  xr_rows1, stage_m1, stage_u1, zero_m, zero_u,
                  msg_acc, u_acc, semg0, semg1, sems0, sems1):
    c = lax.axis_index("c")
    s = lax.axis_index("s")
    sd = (sd_v0, sd_v1)
    dsts = (dsts_v0, dsts_v1)
    xl_rows = (xl_rows0, xl_rows1)
    xr_rows = (xr_rows0, xr_rows1)
    stage_m = (stage_m0, stage_m1)
    stage_u = (stage_u0, stage_u1)
    semg = (semg0, semg1)
    sems = (sems0, sems1)

    zv = jnp.zeros((16,), jnp.float32)

    def zrow(i, carry):
        for k in range(HD // 16):
            zero_m[i, pl.ds(k * 16, 16)] = zv
        zero_u[i, :] = zv
        return carry

    lax.fori_loop(0, 64, zrow, 0)
    rbase = s * RPT

    def zacc(i, carry):
        pltpu.sync_copy(zero_m, msg_acc.at[pl.ds(rbase + i * 64, 64)])
        pltpu.sync_copy(zero_u, u_acc.at[pl.ds(rbase + i * 64, 64)])
        return carry

    lax.fori_loop(0, RPT // 64, zacc, 0)
    pltpu.sync_copy(att_hbm, att_v)
    plsc.subcore_barrier()

    att = [att_v[pl.ds(c * HD + h * 16, 16)] for h in range(H // NC)]
    iota = lax.iota(jnp.int32, 16)
    row_off = jnp.broadcast_to(c * OFFN, (16,)).astype(jnp.int32)
    cbase0 = s * NCH

    def issue_gather(p, ck):
        pltpu.sync_copy(sd_hbm.at[ck], sd[p])
        for j in range(2 * CH // 16):
            sl = pl.ds(j * 16, 16)
            sd[p][sl] = sd[p][sl] + row_off
        pltpu.async_copy(xl_hbm.at[sd[p].at[pl.ds(0, CH)]], xl_rows[p],
                         semg[p])
        pltpu.async_copy(xr_hbm.at[sd[p].at[pl.ds(CH, CH)]], xr_rows[p],
                         semg[p])

    def wait_gather(p):
        pltpu.make_async_copy(xl_hbm.at[sd[p].at[pl.ds(0, CH)]], xl_rows[p],
                              semg[p]).wait()
        pltpu.make_async_copy(xr_hbm.at[sd[p].at[pl.ds(CH, CH)]], xr_rows[p],
                              semg[p]).wait()

    def wait_scatter(p):
        pltpu.make_async_copy(stage_m[p], msg_acc.at[dsts[p]], sems[p]).wait()
        pltpu.make_async_copy(stage_u[p], u_acc.at[dsts[p]], sems[p]).wait()

    def compute(p):
        @plsc.parallel_loop(0, CH, unroll=8)
        def edge(e):
            usl = jnp.zeros((16,), jnp.float32)
            for h in range(H // NC):
                a = xl_rows[p][e, pl.ds(h * 16, 16)]
                bb = xr_rows[p][e, pl.ds(h * 16, 16)]
                v = a + bb
                v = jnp.maximum(v, 0.2 * v)
                lg = jnp.sum(v * att[h])
                u = jnp.exp(jnp.broadcast_to(lg, (16,)))
                stage_m[p][e, pl.ds(h * 16, 16)] = u * a
                usl = jnp.where(iota == h, u, usl)
            stage_u[p][e, :] = usl

    def issue_scatter(p, eb):
        for j in range(CH // 16):
            sl = pl.ds(j * 16, 16)
            dsts[p][sl] = sd[p][pl.ds(CH + j * 16, 16)] - row_off
        pltpu.async_copy(stage_m[p], msg_acc.at[dsts[p]], sems[p], add=True)
        pltpu.async_copy(stage_u[p], u_acc.at[dsts[p]], sems[p], add=True)

    nsup = NCH // 2
    issue_gather(0, cbase0)

    def sup(k2, carry):
        c0 = cbase0 + 2 * k2
        issue_gather(1, c0 + 1)
        wait_gather(0)

        @pl.when(k2 > 0)
        def _():
            wait_scatter(0)

        compute(0)
        issue_scatter(0, c0)

        @pl.when(k2 < nsup - 1)
        def _():
            issue_gather(0, c0 + 2)

        wait_gather(1)

        @pl.when(k2 > 0)
        def _():
            wait_scatter(1)

        compute(1)
        issue_scatter(1, c0 + 1)
        return carry

    lax.fori_loop(0, nsup, sup, 0)
    wait_scatter(0)
    wait_scatter(1)
    plsc.subcore_barrier()
    pltpu.sync_copy(msg_acc.at[pl.ds(rbase, RPT)],
                    msg_out.at[c, pl.ds(rbase, RPT)])
    pltpu.sync_copy(u_acc.at[pl.ds(rbase, RPT)],
                    u_out.at[c, pl.ds(rbase, RPT)])


@functools.cache
def _sc_call():
    return pl.kernel(
        _sc_edge_pass,
        out_type=[jax.ShapeDtypeStruct((NC, NR, HD), jnp.float32),
                  jax.ShapeDtypeStruct((NC, NR, 16), jnp.float32)],
        mesh=plsc.VectorSubcoreMesh(core_axis_name="c", subcore_axis_name="s"),
        compiler_params=pltpu.CompilerParams(needs_layout_passes=False,
                                             use_tc_tiling_on_sc=False),
        scratch_types=(
            [pltpu.VMEM((D,), jnp.float32)]
            + 2 * [pltpu.VMEM((2 * CH,), jnp.int32),
                   pltpu.VMEM((CH,), jnp.int32),
                   pltpu.VMEM((CH, HD), jnp.float32),
                   pltpu.VMEM((CH, HD), jnp.float32),
                   pltpu.VMEM((CH, HD), jnp.float32),
                   pltpu.VMEM((CH, 16), jnp.float32)]
            + [pltpu.VMEM((64, HD), jnp.float32),
               pltpu.VMEM((64, 16), jnp.float32),
               pltpu.VMEM_SHARED((NR, HD), jnp.float32),
               pltpu.VMEM_SHARED((NR, 16), jnp.float32)]
            + 4 * [pltpu.SemaphoreType.DMA]
        ),
    )


def _mm_body(x_ref, w_ref, xl_ref, xr_ref):
    acc = jnp.dot(x_ref[...], w_ref[...], preferred_element_type=jnp.float32)
    xl_ref[0] = acc[:, 0 * HD:1 * HD]
    xl_ref[1] = acc[:, 1 * HD:2 * HD]
    xr_ref[0] = acc[:, 2 * HD:3 * HD]
    xr_ref[1] = acc[:, 3 * HD:4 * HD]


@functools.cache
def _mm_call():
    return pl.pallas_call(
        _mm_body,
        grid=(N // BR,),
        in_specs=[pl.BlockSpec((BR, D), lambda i: (i, 0)),
                  pl.BlockSpec((D, 2 * D), lambda i: (0, 0))],
        out_specs=[pl.BlockSpec((NC, BR, HD), lambda i: (0, i, 0)),
                   pl.BlockSpec((NC, BR, HD), lambda i: (0, i, 0))],
        out_shape=[jax.ShapeDtypeStruct((NC, OFFN, HD), jnp.float32),
                   jax.ShapeDtypeStruct((NC, OFFN, HD), jnp.float32)],
    )


def _norm_block(m_ref, u_ref, b_ref, bmat_ref):
    m = jnp.concatenate([m_ref[0], m_ref[1]], axis=1)
    u8 = jnp.concatenate([u_ref[0][:, :H // NC], u_ref[1][:, :H // NC]],
                         axis=1)
    recip = 1.0 / (u8 + 1e-16)
    r128 = jnp.dot(recip, bmat_ref[...], preferred_element_type=jnp.float32)
    return m * r128 + b_ref[...]


def _comb1_body(m_ref, u_ref, b_ref, bmat_ref, w_ref, xl_ref, xr_ref):
    x1 = _norm_block(m_ref, u_ref, b_ref, bmat_ref)
    x1 = jnp.where(x1 > 0.0, x1, jnp.exp(x1) - 1.0)
    acc = jnp.dot(x1, w_ref[...], preferred_element_type=jnp.float32)
    xl_ref[0] = acc[:, 0 * HD:1 * HD]
    xl_ref[1] = acc[:, 1 * HD:2 * HD]
    xr_ref[0] = acc[:, 2 * HD:3 * HD]
    xr_ref[1] = acc[:, 3 * HD:4 * HD]


@functools.cache
def _comb1_call():
    return pl.pallas_call(
        _comb1_body,
        grid=(N // BR,),
        in_specs=[pl.BlockSpec((NC, BR, HD), lambda i: (0, i, 0)),
                  pl.BlockSpec((NC, BR, 16), lambda i: (0, i, 0)),
                  pl.BlockSpec((1, D), lambda i: (0, 0)),
                  pl.BlockSpec((H, D), lambda i: (0, 0)),
                  pl.BlockSpec((D, 2 * D), lambda i: (0, 0))],
        out_specs=[pl.BlockSpec((NC, BR, HD), lambda i: (0, i, 0)),
                   pl.BlockSpec((NC, BR, HD), lambda i: (0, i, 0))],
        out_shape=[jax.ShapeDtypeStruct((NC, OFFN, HD), jnp.float32),
                   jax.ShapeDtypeStruct((NC, OFFN, HD), jnp.float32)],
    )


def _comb2_body(m_ref, u_ref, b_ref, bmat_ref, h_ref, ls_ref):
    h = _norm_block(m_ref, u_ref, b_ref, bmat_ref)
    mx = jnp.max(h, axis=1, keepdims=True)
    lse = jnp.log(jnp.sum(jnp.exp(h - mx), axis=1, keepdims=True)) + mx
    h_ref[...] = h
    ls_ref[...] = h - lse


@functools.cache
def _comb2_call():
    return pl.pallas_call(
        _comb2_body,
        grid=(N // BR,),
        in_specs=[pl.BlockSpec((NC, BR, HD), lambda i: (0, i, 0)),
                  pl.BlockSpec((NC, BR, 16), lambda i: (0, i, 0)),
                  pl.BlockSpec((1, D), lambda i: (0, 0)),
                  pl.BlockSpec((H, D), lambda i: (0, 0))],
        out_specs=[pl.BlockSpec((BR, D), lambda i: (i, 0)),
                   pl.BlockSpec((BR, D), lambda i: (i, 0))],
        out_shape=[jax.ShapeDtypeStruct((N, D), jnp.float32),
                   jax.ShapeDtypeStruct((N, D), jnp.float32)],
    )


def kernel(x, edge_index, W_l1, W_r1, att1, b1, W_l2, W_r2, att2, b2):
    loops = jnp.arange(N, dtype=jnp.int32)
    src = jnp.concatenate([edge_index[0], loops])
    dst = jnp.concatenate([edge_index[1], loops])
    pad = EP - (E + N)
    zpad = jnp.zeros((pad,), jnp.int32)
    srcg = jnp.concatenate([src, zpad]).reshape(NS * NCH, 1, CH)
    dstg = jnp.concatenate(
        [dst, jnp.full((pad,), N, jnp.int32)]).reshape(NS * NCH, 1, CH)
    sdidx = jnp.concatenate([srcg, dstg], axis=1).reshape(NS * NCH, 2 * CH)

    bmat = jnp.repeat(jnp.eye(H, dtype=jnp.float32), C, axis=1)

    Wc1 = jnp.concatenate([W_l1, W_r1], axis=1)
    Wc2 = jnp.concatenate([W_l2, W_r2], axis=1)

    xl1, xr1 = _mm_call()(x, Wc1)
    msg1, u1 = _sc_call()(xl1.reshape(NC * OFFN, HD),
                          xr1.reshape(NC * OFFN, HD),
                          sdidx, att1.reshape(-1))
    xl2, xr2 = _comb1_call()(msg1, u1, b1.reshape(1, D), bmat, Wc2)
    msg2, u2 = _sc_call()(xl2.reshape(NC * OFFN, HD),
                          xr2.reshape(NC * OFFN, HD),
                          sdidx, att2.reshape(-1))
    h, ls = _comb2_call()(msg2, u2, b2.reshape(1, D), bmat)
    return (h, ls)
